# baseline (speedup 1.0000x reference)
import functools

import jax
import jax.numpy as jnp
from jax import lax
from jax.experimental import pallas as pl
from jax.experimental.pallas import tpu as pltpu
from jax.experimental.pallas import tpu_sc as plsc

N = 32768
D = 128
B = 16
CHUNK = 256
UNROLL = 8
NVEC = D // 16
TCBLK = 512
TCGRID = 33


def _sc_body(values_hbm, ps_hbm, out_hbm,
             ps_v, buf, ebuf, acc_v, merge_v, outbuf_v, shared,
             sem0, sem1):
    c = lax.axis_index("c")
    s = lax.axis_index("s")
    b = 8 + c * 4 + s // 4
    q = s % 4

    pltpu.sync_copy(ps_hbm, ps_v.at[pl.ds(0, B + 1)])
    ps_vec = ps_v[pl.ds(b, 16)]
    start = ps_vec[0]
    end = ps_vec[1]

    quarter = (end - start) >> 2
    lo = jnp.where(q == 0, start, (start + q * quarter) & -8)
    hi = jnp.where(q == 3, end, (start + (q + 1) * quarter) & -8)
    lo_a = (lo + 7) & -8
    hi_a = hi & -8
    n_chunks = (hi_a - lo_a + (CHUNK - 1)) >> 8

    neg_inf = jnp.full((16,), -jnp.inf, dtype=jnp.float32)
    for j in range(NVEC):
        acc_v[pl.ds(16 * j, 16)] = neg_inf

    sems = (sem0, sem1)

    def chunk_off(i):
        off = jnp.minimum(lo_a + i * CHUNK, hi_a - CHUNK)
        return pl.multiple_of(off, 8)

    def start_dma(i, parity):
        pltpu.make_async_copy(
            values_hbm.at[pl.ds(chunk_off(i), CHUNK)],
            buf.at[parity], sems[parity]).start()

    def process(i, parity):
        pltpu.make_async_copy(
            values_hbm.at[pl.ds(0, CHUNK)],
            buf.at[parity], sems[parity]).wait()
        cbuf = buf.at[parity]
        acc = tuple(acc_v[pl.ds(16 * j, 16)] for j in range(NVEC))
        lo_row = (lo_a + i * CHUNK) - chunk_off(i)

        def row_body(it, a):
            r0 = it * UNROLL
            out = []
            for j in range(NVEC):
                vs = [cbuf[r0 + u, pl.ds(16 * j, 16)] for u in range(UNROLL)]
                while len(vs) > 1:
                    vs = [jnp.maximum(vs[i2], vs[i2 + 1])
                          for i2 in range(0, len(vs), 2)]
                out.append(jnp.maximum(a[j], vs[0]))
            return tuple(out)

        acc = lax.fori_loop(lo_row >> 3, CHUNK // UNROLL, row_body, acc)
        for j in range(NVEC):
            acc_v[pl.ds(16 * j, 16)] = acc[j]

    start_dma(0, 0)
    pl.when(1 < n_chunks)(lambda: start_dma(1, 1))

    def pair_body(k, carry):
        process(2 * k, 0)
        pl.when(2 * k + 2 < n_chunks)(lambda: start_dma(2 * k + 2, 0))
        process(2 * k + 1, 1)
        pl.when(2 * k + 3 < n_chunks)(lambda: start_dma(2 * k + 3, 1))
        return carry

    lax.fori_loop(0, n_chunks >> 1, pair_body, 0)
    last = n_chunks - 1
    pl.when((n_chunks & 1) == 1)(lambda: process(last, 0))

    acc = list(acc_v[pl.ds(16 * j, 16)] for j in range(NVEC))

    def edge(acc, off, row_lo, row_hi):
        off = pl.multiple_of(off, 8)
        pltpu.sync_copy(values_hbm.at[pl.ds(off, 8)], ebuf)
        for r in range(8):
            g = off + r
            pred = jnp.logical_and(g >= row_lo, g < row_hi)
            for j in range(NVEC):
                acc[j] = jnp.maximum(
                    acc[j],
                    jnp.where(pred, ebuf[r, pl.ds(16 * j, 16)], neg_inf))
        return acc

    acc = edge(acc, jnp.maximum(lo_a - 8, 0), lo, lo_a)
    acc = edge(acc, jnp.minimum(hi_a, N - 8), hi_a, hi)

    for j in range(NVEC):
        acc_v[pl.ds(16 * j, 16)] = acc[j]
    pltpu.sync_copy(acc_v, shared.at[s])
    plsc.subcore_barrier()

    @pl.when(s == 0)
    def _():
        pltpu.sync_copy(shared, merge_v)
        for k in range(4):
            for j in range(NVEC):
                m01 = jnp.maximum(merge_v[4 * k, pl.ds(16 * j, 16)],
                                  merge_v[4 * k + 1, pl.ds(16 * j, 16)])
                m23 = jnp.maximum(merge_v[4 * k + 2, pl.ds(16 * j, 16)],
                                  merge_v[4 * k + 3, pl.ds(16 * j, 16)])
                outbuf_v[k, pl.ds(16 * j, 16)] = jnp.maximum(m01, m23)
        for k in range(4, 8):
            for j in range(NVEC):
                outbuf_v[k, pl.ds(16 * j, 16)] = neg_inf
        pltpu.sync_copy(outbuf_v, out_hbm.at[c])


def _tc_body(ps_ref, x_ref, o_ref):
    i = pl.program_id(0)
    base = i * TCBLK

    @pl.when(i == 0)
    def _():
        o_ref[...] = jnp.full((8, D), -jnp.inf, dtype=jnp.float32)

    ps8 = ps_ref[8]
    s0 = jnp.int32(0)
    for k in range(1, 8):
        s0 = s0 + (ps_ref[k] <= base).astype(jnp.int32)
    p1 = jnp.clip(ps_ref[s0 + 1], base, base + TCBLK)

    g = base + lax.broadcasted_iota(jnp.int32, (TCBLK, D), 0)
    x = x_ref[...]
    neg = jnp.float32(-jnp.inf)
    max_a = jnp.max(jnp.where(jnp.logical_and(g < p1, g < ps8), x, neg),
                    axis=0)
    max_b = jnp.max(jnp.where(jnp.logical_and(g >= p1, g < ps8), x, neg),
                    axis=0)
    rowi = lax.broadcasted_iota(jnp.int32, (8, D), 0)
    upd_a = jnp.where(rowi == s0, max_a[None, :], neg)
    upd_b = jnp.where(rowi == jnp.minimum(s0 + 1, 7), max_b[None, :], neg)
    o_ref[...] = jnp.maximum(o_ref[...], jnp.maximum(upd_a, upd_b))


@jax.jit
def kernel(values, prefix_sum):
    ps32 = prefix_sum.astype(jnp.int32)

    tc_out = pl.pallas_call(
        _tc_body,
        grid_spec=pltpu.PrefetchScalarGridSpec(
            num_scalar_prefetch=1,
            grid=(TCGRID,),
            in_specs=[pl.BlockSpec((TCBLK, D), lambda i, ps: (i, 0))],
            out_specs=pl.BlockSpec((8, D), lambda i, ps: (0, 0)),
        ),
        out_shape=jax.ShapeDtypeStruct((8, D), jnp.float32),
    )(ps32, values)

    sc_run = functools.partial(
        pl.kernel,
        mesh=plsc.VectorSubcoreMesh(core_axis_name="c", subcore_axis_name="s"),
        out_type=jax.ShapeDtypeStruct((2, 8, D), jnp.float32),
        scratch_types=[
            pltpu.VMEM((32,), jnp.int32),
            pltpu.VMEM((2, CHUNK, D), jnp.float32),
            pltpu.VMEM((8, D), jnp.float32),
            pltpu.VMEM((D,), jnp.float32),
            pltpu.VMEM((16, D), jnp.float32),
            pltpu.VMEM((8, D), jnp.float32),
            pltpu.VMEM_SHARED((16, D), jnp.float32),
            pltpu.SemaphoreType.DMA,
            pltpu.SemaphoreType.DMA,
        ],
    )(_sc_body)
    sc_out = sc_run(values, ps32)

    return jnp.concatenate([tc_out, sc_out[0, :4], sc_out[1, :4]], axis=0)


# speedup vs baseline: 1.1278x; 1.1278x over previous
import functools

import jax
import jax.numpy as jnp
from jax import lax
from jax.experimental import pallas as pl
from jax.experimental.pallas import tpu as pltpu
from jax.experimental.pallas import tpu_sc as plsc

N = 32768
D = 128
B = 16
CHUNK = 256
UNROLL = 8
NVEC = D // 16


def _segment_max_body(values_hbm, ps_hbm, out_hbm,
                      ps_v, buf, ebuf, acc_v, merge_v, outbuf_v, shared,
                      sem0, sem1):
    c = lax.axis_index("c")
    s = lax.axis_index("s")
    b = c * 8 + s // 2
    h = s % 2

    pltpu.sync_copy(ps_hbm, ps_v.at[pl.ds(0, B + 1)])
    ps_vec = ps_v[pl.ds(b, 16)]
    start = ps_vec[0]
    end = ps_vec[1]

    mid = (start + ((end - start) >> 1)) & -8
    lo = jnp.where(h == 0, start, mid)
    hi = jnp.where(h == 0, mid, end)
    lo_a = (lo + 7) & -8
    hi_a = hi & -8
    n_chunks = (hi_a - lo_a + (CHUNK - 1)) >> 8

    neg_inf = jnp.full((16,), -jnp.inf, dtype=jnp.float32)
    for j in range(NVEC):
        acc_v[pl.ds(16 * j, 16)] = neg_inf

    sems = (sem0, sem1)

    def chunk_off(i):
        off = jnp.minimum(lo_a + i * CHUNK, hi_a - CHUNK)
        return pl.multiple_of(off, 8)

    def start_dma(i, parity):
        pltpu.make_async_copy(
            values_hbm.at[pl.ds(chunk_off(i), CHUNK)],
            buf.at[parity], sems[parity]).start()

    def process(i, parity):
        pltpu.make_async_copy(
            values_hbm.at[pl.ds(0, CHUNK)],
            buf.at[parity], sems[parity]).wait()
        cbuf = buf.at[parity]
        acc = tuple(acc_v[pl.ds(16 * j, 16)] for j in range(NVEC))
        lo_row = (lo_a + i * CHUNK) - chunk_off(i)

        def row_body(it, a):
            r0 = it * UNROLL
            out = []
            for j in range(NVEC):
                vs = [cbuf[r0 + u, pl.ds(16 * j, 16)] for u in range(UNROLL)]
                while len(vs) > 1:
                    vs = [jnp.maximum(vs[i], vs[i + 1])
                          for i in range(0, len(vs), 2)]
                out.append(jnp.maximum(a[j], vs[0]))
            return tuple(out)

        acc = lax.fori_loop(lo_row >> 3, CHUNK // UNROLL, row_body, acc)
        for j in range(NVEC):
            acc_v[pl.ds(16 * j, 16)] = acc[j]

    start_dma(0, 0)
    pl.when(1 < n_chunks)(lambda: start_dma(1, 1))

    def pair_body(k, carry):
        process(2 * k, 0)
        pl.when(2 * k + 2 < n_chunks)(lambda: start_dma(2 * k + 2, 0))
        process(2 * k + 1, 1)
        pl.when(2 * k + 3 < n_chunks)(lambda: start_dma(2 * k + 3, 1))
        return carry

    lax.fori_loop(0, n_chunks >> 1, pair_body, 0)
    last = n_chunks - 1
    pl.when((n_chunks & 1) == 1)(lambda: process(last, 0))

    acc = list(acc_v[pl.ds(16 * j, 16)] for j in range(NVEC))

    def edge(acc, off, row_lo, row_hi):
        off = pl.multiple_of(off, 8)
        pltpu.sync_copy(values_hbm.at[pl.ds(off, 8)], ebuf)
        for r in range(8):
            g = off + r
            pred = jnp.logical_and(g >= row_lo, g < row_hi)
            for j in range(NVEC):
                acc[j] = jnp.maximum(
                    acc[j],
                    jnp.where(pred, ebuf[r, pl.ds(16 * j, 16)], neg_inf))
        return acc

    acc = edge(acc, jnp.maximum(lo_a - 8, 0), lo, lo_a)
    acc = edge(acc, jnp.minimum(hi_a, N - 8), hi_a, hi)

    for j in range(NVEC):
        acc_v[pl.ds(16 * j, 16)] = acc[j]
    pltpu.sync_copy(acc_v, shared.at[s])
    plsc.subcore_barrier()

    @pl.when(s == 0)
    def _():
        pltpu.sync_copy(shared, merge_v)
        for k in range(8):
            for j in range(NVEC):
                outbuf_v[k, pl.ds(16 * j, 16)] = jnp.maximum(
                    merge_v[2 * k, pl.ds(16 * j, 16)],
                    merge_v[2 * k + 1, pl.ds(16 * j, 16)])
        base = pl.multiple_of(c * 8, 8)
        pltpu.sync_copy(outbuf_v, out_hbm.at[pl.ds(base, 8)])


@jax.jit
def kernel(values, prefix_sum):
    run = functools.partial(
        pl.kernel,
        mesh=plsc.VectorSubcoreMesh(core_axis_name="c", subcore_axis_name="s"),
        out_type=jax.ShapeDtypeStruct((B, D), jnp.float32),
        scratch_types=[
            pltpu.VMEM((32,), jnp.int32),
            pltpu.VMEM((2, CHUNK, D), jnp.float32),
            pltpu.VMEM((8, D), jnp.float32),
            pltpu.VMEM((D,), jnp.float32),
            pltpu.VMEM((16, D), jnp.float32),
            pltpu.VMEM((8, D), jnp.float32),
            pltpu.VMEM_SHARED((16, D), jnp.float32),
            pltpu.SemaphoreType.DMA,
            pltpu.SemaphoreType.DMA,
        ],
    )(_segment_max_body)
    return run(values, prefix_sum.astype(jnp.int32))
